# baseline (speedup 1.0000x reference)
import functools

import jax
import jax.numpy as jnp
from jax import lax
from jax.experimental import pallas as pl
from jax.experimental.pallas import tpu as pltpu
from jax.experimental.pallas import tpu_sc as plsc

D_MODEL = 64
SCALE = 8.0
NC, NS = 2, 16
NW = NC * NS
C = 128
NBUF = 8


def kernel(x, table):
    B, L = x.shape
    n = B * L
    V = table.shape[0]
    per_w = n // NW
    chunks = per_w // C
    groups = chunks // NBUF
    idx = x.reshape(n) * 2
    table_lin = lax.optimization_barrier(
        jnp.pad(table * SCALE, ((0, 0), (0, D_MODEL)))
    )
    table_lin = table_lin.reshape(2 * V, D_MODEL)
    mesh = plsc.VectorSubcoreMesh(core_axis_name="c", subcore_axis_name="s")

    @functools.partial(
        pl.kernel,
        out_type=jax.ShapeDtypeStruct((n, 2 * D_MODEL), jnp.float32),
        mesh=mesh,
        compiler_params=pltpu.CompilerParams(use_tc_tiling_on_sc=False),
        scratch_types=[
            pltpu.VMEM((per_w,), jnp.int32),
            pltpu.VMEM((NBUF, C, D_MODEL), jnp.float32),
            pltpu.SemaphoreType.DMA((NBUF,)),
            pltpu.SemaphoreType.DMA((NBUF,)),
            pltpu.SemaphoreType.DMA,
        ],
    )
    def gather_scale(table_hbm, idx_hbm, out_hbm, idx_v, rows_v, gsem, ssem, isem):
        wid = lax.axis_index("s") * NC + lax.axis_index("c")
        base = pl.multiple_of(wid * per_w, per_w)
        pltpu.async_copy(idx_hbm.at[pl.ds(base, per_w)], idx_v, isem).wait()

        @pl.loop(0, groups)
        def _(g):
            j0 = g * NBUF
            fired = []
            for b in range(NBUF):
                off = pl.multiple_of((j0 + b) * C, C)

                @pl.when(g > 0)
                def _():
                    pltpu.make_async_copy(
                        rows_v.at[b],
                        out_hbm.at[pl.ds(base + off - NBUF * C, C), pl.ds(0, D_MODEL)],
                        ssem.at[b],
                    ).wait()

                fired.append(
                    pltpu.async_copy(
                        table_hbm.at[idx_v.at[pl.ds(off, C)]],
                        rows_v.at[b],
                        gsem.at[b],
                    )
                )
            for b in range(NBUF):
                off = pl.multiple_of((j0 + b) * C, C)
                fired[b].wait()

                pltpu.async_copy(
                    rows_v.at[b],
                    out_hbm.at[pl.ds(base + off, C), pl.ds(0, D_MODEL)],
                    ssem.at[b],
                )

        for b in range(NBUF):
            off = ((groups - 1) * NBUF + b) * C
            pltpu.make_async_copy(
                rows_v.at[b],
                out_hbm.at[pl.ds(base + off, C), pl.ds(0, D_MODEL)],
                ssem.at[b],
            ).wait()

    out = gather_scale(table_lin, idx)
    return out[:, :D_MODEL].reshape(B, L, D_MODEL)


# speedup vs baseline: 1.1020x; 1.1020x over previous
import functools

import jax
import jax.numpy as jnp
from jax import lax
from jax.experimental import pallas as pl
from jax.experimental.pallas import tpu as pltpu
from jax.experimental.pallas import tpu_sc as plsc

D_MODEL = 64
SCALE = 8.0
NC, NS = 2, 16
NW = NC * NS
C = 128
NBUF = 8


def kernel(x, table):
    B, L = x.shape
    n = B * L
    V = table.shape[0]
    per_w = n // NW
    chunks = per_w // C
    groups = chunks // NBUF
    idx = x.reshape(n) * 2
    table_lin = lax.optimization_barrier(jnp.pad(table, ((0, 0), (0, D_MODEL))))
    table_lin = table_lin.reshape(2 * V, D_MODEL)
    mesh = plsc.VectorSubcoreMesh(core_axis_name="c", subcore_axis_name="s")

    @functools.partial(
        pl.kernel,
        out_type=jax.ShapeDtypeStruct((n, 2 * D_MODEL), jnp.float32),
        mesh=mesh,
        compiler_params=pltpu.CompilerParams(use_tc_tiling_on_sc=False),
        scratch_types=[
            pltpu.VMEM((per_w,), jnp.int32),
            pltpu.VMEM((NBUF, C, D_MODEL), jnp.float32),
            pltpu.SemaphoreType.DMA((NBUF,)),
            pltpu.SemaphoreType.DMA((NBUF,)),
            pltpu.SemaphoreType.DMA,
        ],
    )
    def gather_scale(table_hbm, idx_hbm, out_hbm, idx_v, rows_v, gsem, ssem, isem):
        wid = lax.axis_index("s") * NC + lax.axis_index("c")
        base = pl.multiple_of(wid * per_w, per_w)
        pltpu.async_copy(idx_hbm.at[pl.ds(base, per_w)], idx_v, isem).wait()

        @pl.loop(0, groups)
        def _(g):
            j0 = g * NBUF
            fired = []
            for b in range(NBUF):
                off = pl.multiple_of((j0 + b) * C, C)

                @pl.when(g > 0)
                def _():
                    pltpu.make_async_copy(
                        rows_v.at[b],
                        out_hbm.at[pl.ds(base + off - NBUF * C, C), pl.ds(0, D_MODEL)],
                        ssem.at[b],
                    ).wait()

                fired.append(
                    pltpu.async_copy(
                        table_hbm.at[idx_v.at[pl.ds(off, C)]],
                        rows_v.at[b],
                        gsem.at[b],
                    )
                )
            for b in range(NBUF):
                off = pl.multiple_of((j0 + b) * C, C)
                fired[b].wait()

                @pl.loop(0, C)
                def _(r):
                    for c0 in range(0, D_MODEL, 16):
                        rows_v[b, r, pl.ds(c0, 16)] = (
                            rows_v[b, r, pl.ds(c0, 16)] * SCALE
                        )

                pltpu.async_copy(
                    rows_v.at[b],
                    out_hbm.at[pl.ds(base + off, C), pl.ds(0, D_MODEL)],
                    ssem.at[b],
                )

        for b in range(NBUF):
            off = ((groups - 1) * NBUF + b) * C
            pltpu.make_async_copy(
                rows_v.at[b],
                out_hbm.at[pl.ds(base + off, C), pl.ds(0, D_MODEL)],
                ssem.at[b],
            ).wait()

    out = gather_scale(table_lin, idx)
    return out[:, :D_MODEL].reshape(B, L, D_MODEL)


# speedup vs baseline: 1.1999x; 1.0888x over previous
import functools

import jax
import jax.numpy as jnp
from jax import lax
from jax.experimental import pallas as pl
from jax.experimental.pallas import tpu as pltpu
from jax.experimental.pallas import tpu_sc as plsc

D_MODEL = 64
SCALE = 8.0
NC, NS = 2, 16
NW = NC * NS
C = 128
NBUF = 8


def kernel(x, table):
    B, L = x.shape
    n = B * L
    V = table.shape[0]
    per_w = n // NW
    chunks = per_w // C
    groups = chunks // NBUF
    idx = x.reshape(n) * 2
    table_lin = jnp.pad(table, ((0, 0), (0, D_MODEL)))
    table_lin = table_lin.reshape(2 * V, D_MODEL)
    mesh = plsc.VectorSubcoreMesh(core_axis_name="c", subcore_axis_name="s")

    @functools.partial(
        pl.kernel,
        out_type=jax.ShapeDtypeStruct((n, 2 * D_MODEL), jnp.float32),
        mesh=mesh,
        compiler_params=pltpu.CompilerParams(use_tc_tiling_on_sc=False),
        scratch_types=[
            pltpu.VMEM((per_w,), jnp.int32),
            pltpu.VMEM((NBUF, C, D_MODEL), jnp.float32),
            pltpu.SemaphoreType.DMA((NBUF,)),
            pltpu.SemaphoreType.DMA((NBUF,)),
            pltpu.SemaphoreType.DMA,
        ],
    )
    def gather_scale(table_hbm, idx_hbm, out_hbm, idx_v, rows_v, gsem, ssem, isem):
        wid = lax.axis_index("s") * NC + lax.axis_index("c")
        base = pl.multiple_of(wid * per_w, per_w)
        pltpu.async_copy(idx_hbm.at[pl.ds(base, per_w)], idx_v, isem).wait()

        @pl.loop(0, groups)
        def _(g):
            j0 = g * NBUF
            fired = []
            for b in range(NBUF):
                off = pl.multiple_of((j0 + b) * C, C)

                @pl.when(g > 0)
                def _():
                    pltpu.make_async_copy(
                        rows_v.at[b],
                        out_hbm.at[pl.ds(base + off - NBUF * C, C), pl.ds(0, D_MODEL)],
                        ssem.at[b],
                    ).wait()

                fired.append(
                    pltpu.async_copy(
                        table_hbm.at[idx_v.at[pl.ds(off, C)]],
                        rows_v.at[b],
                        gsem.at[b],
                    )
                )
            for b in range(NBUF):
                off = pl.multiple_of((j0 + b) * C, C)
                fired[b].wait()

                @pl.loop(0, C)
                def _(r):
                    for c0 in range(0, D_MODEL, 16):
                        rows_v[b, r, pl.ds(c0, 16)] = (
                            rows_v[b, r, pl.ds(c0, 16)] * SCALE
                        )

                pltpu.async_copy(
                    rows_v.at[b],
                    out_hbm.at[pl.ds(base + off, C), pl.ds(0, D_MODEL)],
                    ssem.at[b],
                )

        for b in range(NBUF):
            off = ((groups - 1) * NBUF + b) * C
            pltpu.make_async_copy(
                rows_v.at[b],
                out_hbm.at[pl.ds(base + off, C), pl.ds(0, D_MODEL)],
                ssem.at[b],
            ).wait()

    out = gather_scale(table_lin, idx)
    return out[:, :D_MODEL].reshape(B, L, D_MODEL)
